# baseline (speedup 1.0000x reference)
import functools

import jax
import jax.numpy as jnp
from jax import lax
from jax.experimental import pallas as pl
from jax.experimental.pallas import tpu as pltpu
from jax.experimental.pallas import tpu_sc as plsc

VOCAB = 100000
EMBED = 32
BLOCK = 8
BATCH = 128
ROWS = BATCH * BLOCK

NC = 2
NS = 16
NW = NC * NS
ROWS_PER_W = ROWS // NW

VT = 2048
NVT = (VOCAB + VT - 1) // VT
VLAST = VOCAB - (NVT - 1) * VT
KRING = 4
P = 2
BP = BATCH // P


def _emb_body(tok_hbm, idx_hbm, pos_hbm, out_hbm, idx_v, rows_v, pos_v, sem):
    wid = lax.axis_index("s") * NC + lax.axis_index("c")
    base = wid * ROWS_PER_W
    pltpu.sync_copy(idx_hbm.at[pl.ds(base, ROWS_PER_W)], idx_v)
    pltpu.sync_copy(pos_hbm, pos_v)
    pltpu.async_copy(tok_hbm.at[idx_v], rows_v, sem).wait()
    for r in range(ROWS_PER_W):
        for c in range(EMBED // 16):
            sl = pl.ds(c * 16, 16)
            rows_v[r, sl] = rows_v[r, sl] + pos_v[r % BLOCK, sl]
    pltpu.sync_copy(rows_v, out_hbm.at[pl.ds(base, ROWS_PER_W)])


@functools.cache
def _emb_kernel():
    return pl.kernel(
        _emb_body,
        out_type=jax.ShapeDtypeStruct((ROWS, EMBED), jnp.float32),
        mesh=plsc.VectorSubcoreMesh(
            core_axis_name="c", subcore_axis_name="s", num_cores=NC, num_subcores=NS
        ),
        scratch_types=[
            pltpu.VMEM((ROWS_PER_W,), jnp.int32),
            pltpu.VMEM((ROWS_PER_W, EMBED), jnp.float32),
            pltpu.VMEM((BLOCK, EMBED), jnp.float32),
            pltpu.SemaphoreType.DMA,
        ],
        compiler_params=pltpu.CompilerParams(use_tc_tiling_on_sc=False),
    )


def _out_copy(buf, out_hbm, sem_p, col, width, p):
    return pltpu.make_async_copy(
        buf.at[pl.ds(p * BP, BP), :, pl.ds(0, width)],
        out_hbm.at[pl.ds(p * BP, BP), :, pl.ds(col, width)],
        sem_p.at[p],
    )


def _mm_body(x_ref, w_ref, b_ref, out_hbm, r0, r1, r2, r3, tail_v,
             s0, s1, s2, s3, tail_sem):
    j = pl.program_id(0)
    rings = [r0, r1, r2, r3]
    sems = [s0, s1, s2, s3]
    slot = lax.rem(j, KRING)

    for k in range(KRING):
        @pl.when(jnp.logical_and(slot == k, j >= KRING))
        def _wait_prev(k=k):
            for p in range(P):
                _out_copy(rings[k], out_hbm, sems[k], (j - KRING) * VT, VT, p).wait()

    acc = jnp.broadcast_to(b_ref[...], (ROWS, VT)) + x_ref[0, 0]

    for k in range(KRING):
        @pl.when(jnp.logical_and(slot == k, j < NVT - 1))
        def _issue_full(k=k):
            rings[k][...] = acc.reshape(BATCH, BLOCK, VT)
            for p in range(P):
                _out_copy(rings[k], out_hbm, sems[k], j * VT, VT, p).start()

    @pl.when(j == NVT - 1)
    def _issue_last_and_drain():
        tail_v[...] = acc[:, :VLAST].reshape(BATCH, BLOCK, VLAST)
        tail_copy = pltpu.make_async_copy(
            tail_v,
            out_hbm.at[:, :, pl.ds((NVT - 1) * VT, VLAST)],
            tail_sem,
        )
        tail_copy.start()
        for j2 in range(NVT - KRING, NVT - 1):
            for p in range(P):
                _out_copy(rings[j2 % KRING], out_hbm, sems[j2 % KRING],
                          j2 * VT, VT, p).wait()
        tail_copy.wait()


def kernel(idx, tok_table, pos_table, W, b):
    idx_flat = idx.reshape(ROWS).astype(jnp.int32)
    x = _emb_kernel()(tok_table, idx_flat, pos_table)
    logits = pl.pallas_call(
        _mm_body,
        grid=(NVT,),
        in_specs=[
            pl.BlockSpec((ROWS, EMBED), lambda j: (0, 0)),
            pl.BlockSpec((EMBED, VT), lambda j: (0, j)),
            pl.BlockSpec((1, VT), lambda j: (0, j)),
        ],
        out_specs=pl.BlockSpec(memory_space=pl.ANY),
        out_shape=jax.ShapeDtypeStruct((BATCH, BLOCK, VOCAB), jnp.float32),
        scratch_shapes=[
            pltpu.VMEM((BATCH, BLOCK, VT), jnp.float32),
            pltpu.VMEM((BATCH, BLOCK, VT), jnp.float32),
            pltpu.VMEM((BATCH, BLOCK, VT), jnp.float32),
            pltpu.VMEM((BATCH, BLOCK, VT), jnp.float32),
            pltpu.VMEM((BATCH, BLOCK, VLAST), jnp.float32),
            pltpu.SemaphoreType.DMA((P,)),
            pltpu.SemaphoreType.DMA((P,)),
            pltpu.SemaphoreType.DMA((P,)),
            pltpu.SemaphoreType.DMA((P,)),
            pltpu.SemaphoreType.DMA,
        ],
    )(x, W, b.reshape(1, VOCAB))
    return logits


# speedup vs baseline: 1.1463x; 1.1463x over previous
import functools

import jax
import jax.numpy as jnp
from jax import lax
from jax.experimental import pallas as pl
from jax.experimental.pallas import tpu as pltpu
from jax.experimental.pallas import tpu_sc as plsc

VOCAB = 100000
EMBED = 32
BLOCK = 8
BATCH = 128
ROWS = BATCH * BLOCK

NC = 2
NS = 16
NW = NC * NS
ROWS_PER_W = ROWS // NW

RB = 32
NRB = ROWS // RB


def _emb_body(tok_hbm, idx_hbm, pos_hbm, out_hbm, idx_v, rows_v, pos_v, sem):
    wid = lax.axis_index("s") * NC + lax.axis_index("c")
    base = wid * ROWS_PER_W
    pltpu.sync_copy(idx_hbm.at[pl.ds(base, ROWS_PER_W)], idx_v)
    pltpu.sync_copy(pos_hbm, pos_v)
    pltpu.async_copy(tok_hbm.at[idx_v], rows_v, sem).wait()
    for r in range(ROWS_PER_W):
        for c in range(EMBED // 16):
            sl = pl.ds(c * 16, 16)
            rows_v[r, sl] = rows_v[r, sl] + pos_v[r % BLOCK, sl]
    pltpu.sync_copy(rows_v, out_hbm.at[pl.ds(base, ROWS_PER_W)])


@functools.cache
def _emb_kernel():
    return pl.kernel(
        _emb_body,
        out_type=jax.ShapeDtypeStruct((ROWS, EMBED), jnp.float32),
        mesh=plsc.VectorSubcoreMesh(
            core_axis_name="c", subcore_axis_name="s", num_cores=NC, num_subcores=NS
        ),
        scratch_types=[
            pltpu.VMEM((ROWS_PER_W,), jnp.int32),
            pltpu.VMEM((ROWS_PER_W, EMBED), jnp.float32),
            pltpu.VMEM((BLOCK, EMBED), jnp.float32),
            pltpu.SemaphoreType.DMA,
        ],
        compiler_params=pltpu.CompilerParams(use_tc_tiling_on_sc=False),
    )


def _mm_body(x_ref, w_ref, b_ref, o_ref):
    o_ref[...] = (
        jnp.dot(x_ref[...], w_ref[...], preferred_element_type=jnp.float32)
        + b_ref[...]
    )


def kernel(idx, tok_table, pos_table, W, b):
    idx_flat = idx.reshape(ROWS).astype(jnp.int32)
    x = _emb_kernel()(tok_table, idx_flat, pos_table)
    logits = pl.pallas_call(
        _mm_body,
        grid=(NRB,),
        in_specs=[
            pl.BlockSpec((RB, EMBED), lambda i: (i, 0)),
            pl.BlockSpec((EMBED, VOCAB), lambda i: (0, 0)),
            pl.BlockSpec((1, VOCAB), lambda i: (0, 0)),
        ],
        out_specs=pl.BlockSpec((RB, VOCAB), lambda i: (i, 0)),
        out_shape=jax.ShapeDtypeStruct((ROWS, VOCAB), jnp.float32),
    )(x, W, b.reshape(1, VOCAB))
    return logits.reshape(BATCH, BLOCK, VOCAB)


# speedup vs baseline: 1.1466x; 1.0003x over previous
import functools

import jax
import jax.numpy as jnp
from jax import lax
from jax.experimental import pallas as pl
from jax.experimental.pallas import tpu as pltpu
from jax.experimental.pallas import tpu_sc as plsc

VOCAB = 100000
EMBED = 32
BLOCK = 8
BATCH = 128
ROWS = BATCH * BLOCK

NC = 2
NS = 16
NW = NC * NS
ROWS_PER_W = ROWS // NW

RB = 16
NRB = ROWS // RB
KRING = 4
P = 2
RP = RB // P


def _emb_body(tok_hbm, idx_hbm, pos_hbm, out_hbm, idx_v, rows_v, pos_v, sem):
    wid = lax.axis_index("s") * NC + lax.axis_index("c")
    base = wid * ROWS_PER_W
    pltpu.sync_copy(idx_hbm.at[pl.ds(base, ROWS_PER_W)], idx_v)
    pltpu.sync_copy(pos_hbm, pos_v)
    pltpu.async_copy(tok_hbm.at[idx_v], rows_v, sem).wait()
    for r in range(ROWS_PER_W):
        for c in range(EMBED // 16):
            sl = pl.ds(c * 16, 16)
            rows_v[r, sl] = rows_v[r, sl] + pos_v[r % BLOCK, sl]
    pltpu.sync_copy(rows_v, out_hbm.at[pl.ds(base, ROWS_PER_W)])


@functools.cache
def _emb_kernel():
    return pl.kernel(
        _emb_body,
        out_type=jax.ShapeDtypeStruct((ROWS, EMBED), jnp.float32),
        mesh=plsc.VectorSubcoreMesh(
            core_axis_name="c", subcore_axis_name="s", num_cores=NC, num_subcores=NS
        ),
        scratch_types=[
            pltpu.VMEM((ROWS_PER_W,), jnp.int32),
            pltpu.VMEM((ROWS_PER_W, EMBED), jnp.float32),
            pltpu.VMEM((BLOCK, EMBED), jnp.float32),
            pltpu.SemaphoreType.DMA,
        ],
        compiler_params=pltpu.CompilerParams(use_tc_tiling_on_sc=False),
    )


def _out_copy(ring, out_hbm, sems, slot, j, p):
    return pltpu.make_async_copy(
        ring.at[slot, pl.ds(p * RP, RP)],
        out_hbm.at[pl.ds(j * RB + p * RP, RP)],
        sems.at[slot, p],
    )


def _mm_body(x_ref, w_ref, b_ref, out_hbm, ring, sems):
    j = pl.program_id(0)
    slot = lax.rem(j, KRING)

    @pl.when(j >= KRING)
    def _wait_prev():
        for p in range(P):
            _out_copy(ring, out_hbm, sems, slot, j - KRING, p).wait()

    ring[slot] = (
        jnp.dot(x_ref[...], w_ref[...], preferred_element_type=jnp.float32)
        + b_ref[...]
    )
    for p in range(P):
        _out_copy(ring, out_hbm, sems, slot, j, p).start()

    @pl.when(j == NRB - 1)
    def _drain():
        for j2 in range(NRB - KRING, NRB):
            for p in range(P):
                _out_copy(ring, out_hbm, sems, j2 % KRING, j2, p).wait()


def kernel(idx, tok_table, pos_table, W, b):
    idx_flat = idx.reshape(ROWS).astype(jnp.int32)
    x = _emb_kernel()(tok_table, idx_flat, pos_table)
    logits = pl.pallas_call(
        _mm_body,
        grid=(NRB,),
        in_specs=[
            pl.BlockSpec((RB, EMBED), lambda i: (i, 0)),
            pl.BlockSpec((EMBED, VOCAB), lambda i: (0, 0)),
            pl.BlockSpec((1, VOCAB), lambda i: (0, 0)),
        ],
        out_specs=pl.BlockSpec(memory_space=pl.ANY),
        out_shape=jax.ShapeDtypeStruct((ROWS, VOCAB), jnp.float32),
        scratch_shapes=[
            pltpu.VMEM((KRING, RB, VOCAB), jnp.float32),
            pltpu.SemaphoreType.DMA((KRING, P)),
        ],
    )(x, W, b.reshape(1, VOCAB))
    return logits.reshape(BATCH, BLOCK, VOCAB)
